# baseline (speedup 1.0000x reference)
import functools

import jax
import jax.numpy as jnp
from jax import lax
from jax.experimental import pallas as pl
from jax.experimental.pallas import tpu as pltpu
from jax.experimental.pallas import tpu_sc as plsc

N = 10000
E = 320000
D = 128
H = 128
C = 64

NC = 2
NS = 16
NT = NC * NS
EPT = E // NT
K = 64
EPT_PAD = 10240
NCHUNK = EPT_PAD // K
G = 32
NB = 4

N_PAD = 10240
RPT = N_PAD // NS
B_R = 1024


def _make_agg(w, with_deg):
  mesh = plsc.VectorSubcoreMesh(core_axis_name="c", subcore_axis_name="s")
  out_type = [jax.ShapeDtypeStruct((NC, N_PAD, w), jnp.float32)]
  scratch = [
      pltpu.VMEM((G, K), jnp.int32),
      pltpu.VMEM((G, K), jnp.int32),
      [pltpu.VMEM((K, w), jnp.float32) for _ in range(NB)],
      pltpu.VMEM_SHARED((N_PAD, w), jnp.float32),
      [pltpu.SemaphoreType.DMA for _ in range(NB)],
      [pltpu.SemaphoreType.DMA for _ in range(NB)],
  ]
  if with_deg:
    out_type.append(jax.ShapeDtypeStruct((NC, N_PAD), jnp.float32))
    scratch += [
        pltpu.VMEM((K,), jnp.float32),
        pltpu.VMEM_SHARED((N_PAD,), jnp.float32),
        [pltpu.SemaphoreType.DMA for _ in range(NB)],
    ]

  def body(*refs):
    if with_deg:
      (zrows, src, dst, p, ones, dzer, agg_out, deg_out,
       s_idx, d_idx, rows, acc, gsem, ssem, ones_v, dacc, dsem) = refs
    else:
      (zrows, src, dst, p, agg_out,
       s_idx, d_idx, rows, acc, gsem, ssem) = refs
    c = lax.axis_index("c")
    s = lax.axis_index("s")
    tid = c * NS + s
    row0 = s * RPT

    pltpu.sync_copy(zrows, acc.at[pl.ds(row0, RPT)])
    if with_deg:
      pltpu.sync_copy(ones, ones_v)
      pltpu.sync_copy(dzer, dacc.at[pl.ds(row0, RPT)])
    plsc.subcore_barrier()

    def group(grp, carry):
      base = pl.multiple_of(grp * G, 8)
      pltpu.sync_copy(src.at[tid, pl.ds(base, G)], s_idx)
      pltpu.sync_copy(dst.at[tid, pl.ds(base, G)], d_idx)
      gd = [None] * G
      sd = [None] * G
      dd = [None] * G
      gd[0] = pltpu.async_copy(p.at[s_idx.at[0]], rows[0], gsem[0])
      for k in range(G):
        b = k % NB
        if k >= NB - 1:
          sd[k - NB + 1].wait()
          if with_deg:
            dd[k - NB + 1].wait()
        if k + 1 < G:
          bn = (k + 1) % NB
          gd[k + 1] = pltpu.async_copy(p.at[s_idx.at[k + 1]], rows[bn],
                                       gsem[bn])
        gd[k].wait()
        sd[k] = pltpu.async_copy(rows[b], acc.at[d_idx.at[k]], ssem[b],
                                 add=True)
        if with_deg:
          dd[k] = pltpu.async_copy(ones_v, dacc.at[d_idx.at[k]], dsem[b],
                                   add=True)
      for k in range(G - NB + 1, G):
        sd[k].wait()
        if with_deg:
          dd[k].wait()
      return carry

    lax.fori_loop(0, NCHUNK // G, group, 0)
    plsc.subcore_barrier()

    pltpu.sync_copy(acc.at[pl.ds(row0, RPT)], agg_out.at[c, pl.ds(row0, RPT)])
    if with_deg:
      pltpu.sync_copy(dacc.at[pl.ds(row0, RPT)], deg_out.at[c, pl.ds(row0, RPT)])

  return pl.kernel(body, out_type=tuple(out_type), mesh=mesh,
                   scratch_types=scratch)


def _entry_body(x, wn, ws, b, p, s):
  h = x[...]
  p[...] = jnp.dot(h, wn[...], preferred_element_type=jnp.float32)
  s[...] = jnp.dot(h, ws[...], preferred_element_type=jnp.float32) + b[...]


def _mid_body(sp, agg, deg, wn, ws, b, p, s):
  d = jnp.maximum(deg[:, 0:1] + deg[:, 1:2], 1.0)
  h = jnp.maximum(sp[...] + (agg[0] + agg[1]) / d, 0.0)
  p[...] = jnp.dot(h, wn[...], preferred_element_type=jnp.float32)
  s[...] = jnp.dot(h, ws[...], preferred_element_type=jnp.float32) + b[...]


def _fin_body(sp, agg, deg, out):
  d = jnp.maximum(deg[:, 0:1] + deg[:, 1:2], 1.0)
  out[...] = sp[...] + (agg[0] + agg[1]) / d


def _rows(w):
  return pl.BlockSpec((B_R, w), lambda i: (i, 0))


def _full(shape):
  return pl.BlockSpec(shape, lambda i: tuple(0 for _ in shape))


_GRID = N_PAD // B_R


def _entry_call(x, wn, ws, b, w_out):
  return pl.pallas_call(
      _entry_body,
      grid=(_GRID,),
      in_specs=[_rows(128), _full((128, w_out)), _full((128, w_out)),
                _full((1, w_out))],
      out_specs=[_rows(w_out), _rows(w_out)],
      out_shape=[jax.ShapeDtypeStruct((N_PAD, w_out), jnp.float32)] * 2,
  )(x, wn, ws, b)


def _mid_call(sp, agg, deg, wn, ws, b, w_in, w_out):
  return pl.pallas_call(
      _mid_body,
      grid=(_GRID,),
      in_specs=[_rows(w_in),
                pl.BlockSpec((NC, B_R, w_in), lambda i: (0, i, 0)),
                pl.BlockSpec((B_R, NC), lambda i: (i, 0)),
                _full((w_in, w_out)), _full((w_in, w_out)), _full((1, w_out))],
      out_specs=[_rows(w_out), _rows(w_out)],
      out_shape=[jax.ShapeDtypeStruct((N_PAD, w_out), jnp.float32)] * 2,
  )(sp, agg, deg, wn, ws, b)


def _fin_call(sp, agg, deg, w):
  return pl.pallas_call(
      _fin_body,
      grid=(_GRID,),
      in_specs=[_rows(w),
                pl.BlockSpec((NC, B_R, w), lambda i: (0, i, 0)),
                pl.BlockSpec((B_R, NC), lambda i: (i, 0))],
      out_specs=_rows(w),
      out_shape=jax.ShapeDtypeStruct((N_PAD, w), jnp.float32),
  )(sp, agg, deg)


def kernel(x, edge_index, Ws0, Wn0, b0, Ws1, Wn1, b1, Ws2, Wn2, b2):
  pad = EPT_PAD - EPT
  src = jnp.pad(edge_index[0].reshape(NT, EPT), ((0, 0), (0, pad)))
  src = src.reshape(NT, NCHUNK, K)
  dpad = N + jnp.arange(pad, dtype=jnp.int32) % (N_PAD - N)
  dst = jnp.concatenate(
      [edge_index[1].reshape(NT, EPT), jnp.broadcast_to(dpad, (NT, pad))],
      axis=1)
  dst = dst.reshape(NT, NCHUNK, K)
  xp = jnp.concatenate([x, jnp.zeros((N_PAD - N, D), jnp.float32)], axis=0)

  zc = jnp.zeros((128, 64), jnp.float32)
  Wn2p = jnp.concatenate([Wn2, zc], axis=1)
  Ws2p = jnp.concatenate([Ws2, zc], axis=1)
  b2p = jnp.concatenate([b2, jnp.zeros((64,), jnp.float32)])

  z128 = jnp.zeros((RPT, 128), jnp.float32)
  dzer = jnp.zeros((RPT,), jnp.float32)
  ones_k = jnp.ones((K,), jnp.float32)

  agg128_deg = _make_agg(128, True)
  agg128 = _make_agg(128, False)

  p0, s0 = _entry_call(xp, Wn0, Ws0, b0.reshape(1, -1), 128)
  agg0, degp = agg128_deg(z128, src, dst, p0, ones_k, dzer)
  degT = degp.T

  p1, s1 = _mid_call(s0, agg0, degT, Wn1, Ws1, b1.reshape(1, -1), 128, 128)
  (agg1,) = agg128(z128, src, dst, p1)

  p2, s2 = _mid_call(s1, agg1, degT, Wn2p, Ws2p, b2p.reshape(1, -1), 128, 128)
  (agg2,) = agg128(z128, src, dst, p2)

  out = _fin_call(s2, agg2, degT, 128)
  return out[:N, :C]


# speedup vs baseline: 1.8441x; 1.8441x over previous
import functools

import jax
import jax.numpy as jnp
from jax import lax
from jax.experimental import pallas as pl
from jax.experimental.pallas import tpu as pltpu
from jax.experimental.pallas import tpu_sc as plsc

N = 10000
E = 320000
D = 128
H = 128
C = 64

NC = 2
NS = 16
NT = NC * NS
EPT = E // NT
K = 80
NCHUNK = EPT // K

N_PAD = 10240
RPT = N_PAD // NS
B_R = 1024


def _make_agg(w, with_deg):
  mesh = plsc.VectorSubcoreMesh(core_axis_name="c", subcore_axis_name="s")
  out_type = [jax.ShapeDtypeStruct((NC, N_PAD, w), jnp.float32)]
  scratch = [
      pltpu.VMEM((NCHUNK, K), jnp.int32),
      pltpu.VMEM((NCHUNK, K), jnp.int32),
      pltpu.VMEM((K, w), jnp.float32),
      pltpu.VMEM_SHARED((N_PAD, w), jnp.float32),
      pltpu.SemaphoreType.DMA,
  ]
  if with_deg:
    out_type.append(jax.ShapeDtypeStruct((NC, N_PAD), jnp.float32))
    scratch += [
        pltpu.VMEM((K,), jnp.float32),
        pltpu.VMEM_SHARED((N_PAD,), jnp.float32),
    ]

  def body(*refs):
    if with_deg:
      (zrows, src, dst, p, ones, dzer, agg_out, deg_out,
       s_idx, d_idx, rows0, acc, sem0, ones_v, dacc) = refs
    else:
      (zrows, src, dst, p, agg_out,
       s_idx, d_idx, rows0, acc, sem0) = refs
    c = lax.axis_index("c")
    s = lax.axis_index("s")
    tid = c * NS + s
    row0 = s * RPT

    pltpu.sync_copy(src.at[tid], s_idx)
    pltpu.sync_copy(dst.at[tid], d_idx)
    pltpu.sync_copy(zrows, acc.at[pl.ds(row0, RPT)])
    if with_deg:
      pltpu.sync_copy(ones, ones_v)
      pltpu.sync_copy(dzer, dacc.at[pl.ds(row0, RPT)])
    plsc.subcore_barrier()

    def chunk(j, carry):
      pltpu.async_copy(p.at[s_idx.at[j]], rows0, sem0).wait()
      pltpu.sync_copy(rows0, acc.at[d_idx.at[j]], add=True)
      if with_deg:
        pltpu.sync_copy(ones_v, dacc.at[d_idx.at[j]], add=True)
      return carry

    lax.fori_loop(0, NCHUNK, chunk, 0)
    plsc.subcore_barrier()

    pltpu.sync_copy(acc.at[pl.ds(row0, RPT)], agg_out.at[c, pl.ds(row0, RPT)])
    if with_deg:
      pltpu.sync_copy(dacc.at[pl.ds(row0, RPT)], deg_out.at[c, pl.ds(row0, RPT)])

  return pl.kernel(body, out_type=tuple(out_type), mesh=mesh,
                   scratch_types=scratch)


def _entry_body(x, wn, ws, b, p, s):
  h = x[...]
  p[...] = jnp.dot(h, wn[...], preferred_element_type=jnp.float32)
  s[...] = jnp.dot(h, ws[...], preferred_element_type=jnp.float32) + b[...]


def _mid_body(sp, agg, deg, wn, ws, b, p, s):
  d = jnp.maximum(deg[:, 0:1] + deg[:, 1:2], 1.0)
  h = jnp.maximum(sp[...] + (agg[0] + agg[1]) / d, 0.0)
  p[...] = jnp.dot(h, wn[...], preferred_element_type=jnp.float32)
  s[...] = jnp.dot(h, ws[...], preferred_element_type=jnp.float32) + b[...]


def _fin_body(sp, agg, deg, out):
  d = jnp.maximum(deg[:, 0:1] + deg[:, 1:2], 1.0)
  out[...] = sp[...] + (agg[0] + agg[1]) / d


def _rows(w):
  return pl.BlockSpec((B_R, w), lambda i: (i, 0))


def _full(shape):
  return pl.BlockSpec(shape, lambda i: tuple(0 for _ in shape))


_GRID = N_PAD // B_R


def _entry_call(x, wn, ws, b, w_out):
  return pl.pallas_call(
      _entry_body,
      grid=(_GRID,),
      in_specs=[_rows(128), _full((128, w_out)), _full((128, w_out)),
                _full((1, w_out))],
      out_specs=[_rows(w_out), _rows(w_out)],
      out_shape=[jax.ShapeDtypeStruct((N_PAD, w_out), jnp.float32)] * 2,
  )(x, wn, ws, b)


def _mid_call(sp, agg, deg, wn, ws, b, w_in, w_out):
  return pl.pallas_call(
      _mid_body,
      grid=(_GRID,),
      in_specs=[_rows(w_in),
                pl.BlockSpec((NC, B_R, w_in), lambda i: (0, i, 0)),
                pl.BlockSpec((B_R, NC), lambda i: (i, 0)),
                _full((w_in, w_out)), _full((w_in, w_out)), _full((1, w_out))],
      out_specs=[_rows(w_out), _rows(w_out)],
      out_shape=[jax.ShapeDtypeStruct((N_PAD, w_out), jnp.float32)] * 2,
  )(sp, agg, deg, wn, ws, b)


def _fin_call(sp, agg, deg, w):
  return pl.pallas_call(
      _fin_body,
      grid=(_GRID,),
      in_specs=[_rows(w),
                pl.BlockSpec((NC, B_R, w), lambda i: (0, i, 0)),
                pl.BlockSpec((B_R, NC), lambda i: (i, 0))],
      out_specs=_rows(w),
      out_shape=jax.ShapeDtypeStruct((N_PAD, w), jnp.float32),
  )(sp, agg, deg)


def kernel(x, edge_index, Ws0, Wn0, b0, Ws1, Wn1, b1, Ws2, Wn2, b2):
  src = edge_index[0].reshape(NT, NCHUNK, K)
  dst = edge_index[1].reshape(NT, NCHUNK, K)
  xp = jnp.concatenate([x, jnp.zeros((N_PAD - N, D), jnp.float32)], axis=0)

  zc = jnp.zeros((128, 64), jnp.float32)
  Wn2p = jnp.concatenate([Wn2, zc], axis=1)
  Ws2p = jnp.concatenate([Ws2, zc], axis=1)
  b2p = jnp.concatenate([b2, jnp.zeros((64,), jnp.float32)])

  z128 = jnp.zeros((RPT, 128), jnp.float32)
  dzer = jnp.zeros((RPT,), jnp.float32)
  ones_k = jnp.ones((K,), jnp.float32)

  agg128_deg = _make_agg(128, True)
  agg128 = _make_agg(128, False)

  p0, s0 = _entry_call(xp, Wn0, Ws0, b0.reshape(1, -1), 128)
  agg0, degp = agg128_deg(z128, src, dst, p0, ones_k, dzer)
  degT = degp.T

  p1, s1 = _mid_call(s0, agg0, degT, Wn1, Ws1, b1.reshape(1, -1), 128, 128)
  (agg1,) = agg128(z128, src, dst, p1)

  p2, s2 = _mid_call(s1, agg1, degT, Wn2p, Ws2p, b2p.reshape(1, -1), 128, 128)
  (agg2,) = agg128(z128, src, dst, p2)

  out = _fin_call(s2, agg2, degT, 128)
  return out[:N, :C]


# speedup vs baseline: 1.9693x; 1.0679x over previous
import functools

import jax
import jax.numpy as jnp
from jax import lax
from jax.experimental import pallas as pl
from jax.experimental.pallas import tpu as pltpu
from jax.experimental.pallas import tpu_sc as plsc

N = 10000
E = 320000
D = 128
H = 128
C = 64

NC = 2
NS = 16
NT = NC * NS
EPT = E // NT
K = 80
NCHUNK = EPT // K

N_PAD = 10240
RPT = N_PAD // NS
B_R = 1024


def _make_agg(w, with_deg, tc_tiling=True):
  mesh = plsc.VectorSubcoreMesh(core_axis_name="c", subcore_axis_name="s")
  out_type = [jax.ShapeDtypeStruct((NC, N_PAD, w), jnp.float32)]
  scratch = [
      pltpu.VMEM((NCHUNK, K), jnp.int32),
      pltpu.VMEM((NCHUNK, K), jnp.int32),
      pltpu.VMEM((K, w), jnp.float32),
      pltpu.VMEM_SHARED((N_PAD, w), jnp.float32),
      pltpu.SemaphoreType.DMA,
  ]
  if with_deg:
    out_type.append(jax.ShapeDtypeStruct((NC, N_PAD), jnp.float32))
    scratch += [
        pltpu.VMEM((K,), jnp.float32),
        pltpu.VMEM_SHARED((N_PAD,), jnp.float32),
    ]

  def body(*refs):
    if with_deg:
      (zrows, src, dst, p, ones, dzer, agg_out, deg_out,
       s_idx, d_idx, rows0, acc, sem0, ones_v, dacc) = refs
    else:
      (zrows, src, dst, p, agg_out,
       s_idx, d_idx, rows0, acc, sem0) = refs
    c = lax.axis_index("c")
    s = lax.axis_index("s")
    tid = c * NS + s
    row0 = s * RPT

    pltpu.sync_copy(src.at[tid], s_idx)
    pltpu.sync_copy(dst.at[tid], d_idx)
    pltpu.sync_copy(zrows, acc.at[pl.ds(row0, RPT)])
    if with_deg:
      pltpu.sync_copy(ones, ones_v)
      pltpu.sync_copy(dzer, dacc.at[pl.ds(row0, RPT)])
    plsc.subcore_barrier()

    def chunk(j, carry):
      pltpu.async_copy(p.at[s_idx.at[j]], rows0, sem0).wait()
      pltpu.sync_copy(rows0, acc.at[d_idx.at[j]], add=True)
      if with_deg:
        pltpu.sync_copy(ones_v, dacc.at[d_idx.at[j]], add=True)
      return carry

    lax.fori_loop(0, NCHUNK, chunk, 0)
    plsc.subcore_barrier()

    pltpu.sync_copy(acc.at[pl.ds(row0, RPT)], agg_out.at[c, pl.ds(row0, RPT)])
    if with_deg:
      pltpu.sync_copy(dacc.at[pl.ds(row0, RPT)], deg_out.at[c, pl.ds(row0, RPT)])

  return pl.kernel(body, out_type=tuple(out_type), mesh=mesh,
                   scratch_types=scratch,
                   compiler_params=pltpu.CompilerParams(
                       use_tc_tiling_on_sc=tc_tiling))


def _entry_body(x, wn, ws, b, p, s):
  h = x[...]
  p[...] = jnp.dot(h, wn[...], preferred_element_type=jnp.float32)
  s[...] = jnp.dot(h, ws[...], preferred_element_type=jnp.float32) + b[...]


def _mid_body(sp, agg, deg, wn, ws, b, p, s):
  d = jnp.maximum(deg[:, 0:1] + deg[:, 1:2], 1.0)
  h = jnp.maximum(sp[...] + (agg[0] + agg[1]) / d, 0.0)
  p[...] = jnp.dot(h, wn[...], preferred_element_type=jnp.float32)
  s[...] = jnp.dot(h, ws[...], preferred_element_type=jnp.float32) + b[...]


def _fin_body(sp, agg, deg, out):
  d = jnp.maximum(deg[:, 0:1] + deg[:, 1:2], 1.0)
  out[...] = sp[...] + (agg[0] + agg[1]) / d


def _rows(w):
  return pl.BlockSpec((B_R, w), lambda i: (i, 0))


def _full(shape):
  return pl.BlockSpec(shape, lambda i: tuple(0 for _ in shape))


_GRID = N_PAD // B_R


def _entry_call(x, wn, ws, b, w_out):
  return pl.pallas_call(
      _entry_body,
      grid=(_GRID,),
      in_specs=[_rows(128), _full((128, w_out)), _full((128, w_out)),
                _full((1, w_out))],
      out_specs=[_rows(w_out), _rows(w_out)],
      out_shape=[jax.ShapeDtypeStruct((N_PAD, w_out), jnp.float32)] * 2,
  )(x, wn, ws, b)


def _mid_call(sp, agg, deg, wn, ws, b, w_in, w_out):
  return pl.pallas_call(
      _mid_body,
      grid=(_GRID,),
      in_specs=[_rows(w_in),
                pl.BlockSpec((NC, B_R, w_in), lambda i: (0, i, 0)),
                pl.BlockSpec((B_R, NC), lambda i: (i, 0)),
                _full((w_in, w_out)), _full((w_in, w_out)), _full((1, w_out))],
      out_specs=[_rows(w_out), _rows(w_out)],
      out_shape=[jax.ShapeDtypeStruct((N_PAD, w_out), jnp.float32)] * 2,
  )(sp, agg, deg, wn, ws, b)


def _fin_call(sp, agg, deg, w):
  return pl.pallas_call(
      _fin_body,
      grid=(_GRID,),
      in_specs=[_rows(w),
                pl.BlockSpec((NC, B_R, w), lambda i: (0, i, 0)),
                pl.BlockSpec((B_R, NC), lambda i: (i, 0))],
      out_specs=_rows(w),
      out_shape=jax.ShapeDtypeStruct((N_PAD, w), jnp.float32),
  )(sp, agg, deg)


def kernel(x, edge_index, Ws0, Wn0, b0, Ws1, Wn1, b1, Ws2, Wn2, b2):
  src = edge_index[0].reshape(NT, NCHUNK, K)
  dst = edge_index[1].reshape(NT, NCHUNK, K)
  xp = jnp.concatenate([x, jnp.zeros((N_PAD - N, D), jnp.float32)], axis=0)

  z128 = jnp.zeros((RPT, 128), jnp.float32)
  z64 = jnp.zeros((RPT, 64), jnp.float32)
  dzer = jnp.zeros((RPT,), jnp.float32)
  ones_k = jnp.ones((K,), jnp.float32)

  agg128_deg = _make_agg(128, True)
  agg128 = _make_agg(128, False)
  agg64 = _make_agg(64, False, tc_tiling=False)

  p0, s0 = _entry_call(xp, Wn0, Ws0, b0.reshape(1, -1), 128)
  agg0, degp = agg128_deg(z128, src, dst, p0, ones_k, dzer)
  degT = degp.T

  p1, s1 = _mid_call(s0, agg0, degT, Wn1, Ws1, b1.reshape(1, -1), 128, 128)
  (agg1,) = agg128(z128, src, dst, p1)

  p2, s2 = _mid_call(s1, agg1, degT, Wn2, Ws2, b2.reshape(1, -1), 128, 64)
  (agg2,) = agg64(z64, src, dst, p2)

  out = _fin_call(s2, agg2, degT, 64)
  return out[:N]


# speedup vs baseline: 1.9818x; 1.0064x over previous
import functools

import jax
import jax.numpy as jnp
from jax import lax
from jax.experimental import pallas as pl
from jax.experimental.pallas import tpu as pltpu
from jax.experimental.pallas import tpu_sc as plsc

N = 10000
E = 320000
D = 128
H = 128
C = 64

NC = 2
NS = 16
NT = NC * NS
EPT = E // NT
K = 80
NCHUNK = EPT // K

N_PAD = 10240
RPT = N_PAD // NS
B_R = 1024


def _make_agg(w, with_deg, tc_tiling=True):
  mesh = plsc.VectorSubcoreMesh(core_axis_name="c", subcore_axis_name="s")
  out_type = [jax.ShapeDtypeStruct((NC, N_PAD, w), jnp.float32)]
  scratch = [
      pltpu.VMEM((NCHUNK, K), jnp.int32),
      pltpu.VMEM((NCHUNK, K), jnp.int32),
      pltpu.VMEM((K, w), jnp.float32),
      pltpu.VMEM_SHARED((N_PAD, w), jnp.float32),
      pltpu.SemaphoreType.DMA,
  ]
  if with_deg:
    out_type.append(jax.ShapeDtypeStruct((NC, N_PAD), jnp.float32))
    scratch += [
        pltpu.VMEM((K,), jnp.float32),
        pltpu.VMEM_SHARED((N_PAD,), jnp.float32),
    ]

  def body(*refs):
    if with_deg:
      (zrows, src, dst, p, ones, dzer, agg_out, deg_out,
       s_idx, d_idx, rows0, acc, sem0, ones_v, dacc) = refs
    else:
      (zrows, src, dst, p, agg_out,
       s_idx, d_idx, rows0, acc, sem0) = refs
    c = lax.axis_index("c")
    s = lax.axis_index("s")
    tid = c * NS + s
    row0 = s * RPT

    pltpu.sync_copy(src.at[tid], s_idx)
    pltpu.sync_copy(dst.at[tid], d_idx)
    pltpu.sync_copy(zrows, acc.at[pl.ds(row0, RPT)])
    if with_deg:
      pltpu.sync_copy(ones, ones_v)
      pltpu.sync_copy(dzer, dacc.at[pl.ds(row0, RPT)])
    plsc.subcore_barrier()

    def chunk(j, carry):
      pltpu.async_copy(p.at[s_idx.at[j]], rows0, sem0).wait()
      pltpu.sync_copy(rows0, acc.at[d_idx.at[j]], add=True)
      if with_deg:
        pltpu.sync_copy(ones_v, dacc.at[d_idx.at[j]], add=True)
      return carry

    lax.fori_loop(0, NCHUNK, chunk, 0)
    plsc.subcore_barrier()

    pltpu.sync_copy(acc.at[pl.ds(row0, RPT)], agg_out.at[c, pl.ds(row0, RPT)])
    if with_deg:
      pltpu.sync_copy(dacc.at[pl.ds(row0, RPT)], deg_out.at[c, pl.ds(row0, RPT)])

  return pl.kernel(body, out_type=tuple(out_type), mesh=mesh,
                   scratch_types=scratch,
                   compiler_params=pltpu.CompilerParams(
                       use_tc_tiling_on_sc=tc_tiling))


def _entry_body(x, wn, ws, b, p, s):
  h = x[...]
  p[...] = jnp.dot(h, wn[...], preferred_element_type=jnp.float32)
  s[...] = jnp.dot(h, ws[...], preferred_element_type=jnp.float32) + b[...]


def _mid_body(sp, agg, deg, wn, ws, b, p, s):
  d = jnp.maximum(deg[:, 0:1] + deg[:, 1:2], 1.0)
  h = jnp.maximum(sp[...] + (agg[0] + agg[1]) / d, 0.0)
  p[...] = jnp.dot(h, wn[...], preferred_element_type=jnp.float32)
  s[...] = jnp.dot(h, ws[...], preferred_element_type=jnp.float32) + b[...]


def _fin_body(sp, agg, deg, out):
  d = jnp.maximum(deg[:, 0:1] + deg[:, 1:2], 1.0)
  out[...] = sp[...] + (agg[0] + agg[1]) / d


def _rows(w):
  return pl.BlockSpec((B_R, w), lambda i: (i, 0))


def _full(shape):
  return pl.BlockSpec(shape, lambda i: tuple(0 for _ in shape))


_GRID = N_PAD // B_R


def _entry_call(x, wn, ws, b, w_out):
  return pl.pallas_call(
      _entry_body,
      grid=(_GRID,),
      in_specs=[_rows(128), _full((128, w_out)), _full((128, w_out)),
                _full((1, w_out))],
      out_specs=[_rows(w_out), _rows(w_out)],
      out_shape=[jax.ShapeDtypeStruct((N_PAD, w_out), jnp.float32)] * 2,
  )(x, wn, ws, b)


def _mid_call(sp, agg, deg, wn, ws, b, w_in, w_out):
  return pl.pallas_call(
      _mid_body,
      grid=(_GRID,),
      in_specs=[_rows(w_in),
                pl.BlockSpec((NC, B_R, w_in), lambda i: (0, i, 0)),
                pl.BlockSpec((B_R, NC), lambda i: (i, 0)),
                _full((w_in, w_out)), _full((w_in, w_out)), _full((1, w_out))],
      out_specs=[_rows(w_out), _rows(w_out)],
      out_shape=[jax.ShapeDtypeStruct((N_PAD, w_out), jnp.float32)] * 2,
  )(sp, agg, deg, wn, ws, b)


def _fin_call(sp, agg, deg, w):
  return pl.pallas_call(
      _fin_body,
      grid=(_GRID,),
      in_specs=[_rows(w),
                pl.BlockSpec((NC, B_R, w), lambda i: (0, i, 0)),
                pl.BlockSpec((B_R, NC), lambda i: (i, 0))],
      out_specs=_rows(w),
      out_shape=jax.ShapeDtypeStruct((N_PAD, w), jnp.float32),
  )(sp, agg, deg)


def kernel(x, edge_index, Ws0, Wn0, b0, Ws1, Wn1, b1, Ws2, Wn2, b2):
  src = edge_index[0].reshape(NT, NCHUNK, K)
  dst = edge_index[1].reshape(NT, NCHUNK, K)
  xp = jnp.concatenate([x, jnp.zeros((N_PAD - N, D), jnp.float32)], axis=0)

  z128 = jnp.zeros((RPT, 128), jnp.float32)
  z64 = jnp.zeros((RPT, 64), jnp.float32)
  dzer = jnp.zeros((RPT,), jnp.float32)
  ones_k = jnp.ones((K,), jnp.float32)

  agg128_deg = _make_agg(128, True, tc_tiling=False)
  agg128 = _make_agg(128, False, tc_tiling=False)
  agg64 = _make_agg(64, False, tc_tiling=False)

  p0, s0 = _entry_call(xp, Wn0, Ws0, b0.reshape(1, -1), 128)
  agg0, degp = agg128_deg(z128, src, dst, p0, ones_k, dzer)
  degT = degp.T

  p1, s1 = _mid_call(s0, agg0, degT, Wn1, Ws1, b1.reshape(1, -1), 128, 128)
  (agg1,) = agg128(z128, src, dst, p1)

  p2, s2 = _mid_call(s1, agg1, degT, Wn2, Ws2, b2.reshape(1, -1), 128, 64)
  (agg2,) = agg64(z64, src, dst, p2)

  out = _fin_call(s2, agg2, degT, 64)
  return out[:N]


# speedup vs baseline: 2.4678x; 1.2452x over previous
import functools

import jax
import jax.numpy as jnp
from jax import lax
from jax.experimental import pallas as pl
from jax.experimental.pallas import tpu as pltpu
from jax.experimental.pallas import tpu_sc as plsc

N = 10000
E = 320000
D = 128
H = 128
C = 64

NC = 2
NS = 16
NT = NC * NS
EPT = E // NT
K = 80
NCHUNK = EPT // K

N_PAD = 10240
RPT = N_PAD // NS
B_R = 1024


def _make_agg(w, with_deg, tc_tiling=True):
  mesh = plsc.VectorSubcoreMesh(core_axis_name="c", subcore_axis_name="s")
  out_type = [jax.ShapeDtypeStruct((NC, N_PAD, w), jnp.float32)]
  scratch = [
      pltpu.VMEM((NCHUNK, K), jnp.int32),
      pltpu.VMEM((NCHUNK, K), jnp.int32),
      pltpu.VMEM((K, w), jnp.float32),
      pltpu.VMEM((K, w), jnp.float32),
      pltpu.VMEM_SHARED((N_PAD, w), jnp.float32),
      pltpu.SemaphoreType.DMA,
      pltpu.SemaphoreType.DMA,
  ]
  if with_deg:
    out_type.append(jax.ShapeDtypeStruct((NC, N_PAD), jnp.float32))
    scratch += [
        pltpu.VMEM((K,), jnp.float32),
        pltpu.VMEM_SHARED((N_PAD,), jnp.float32),
    ]

  def body(*refs):
    if with_deg:
      (zrows, src, dst, p, ones, dzer, agg_out, deg_out,
       s_idx, d_idx, rows0, rows1, acc, sem0, sem1, ones_v, dacc) = refs
    else:
      (zrows, src, dst, p, agg_out,
       s_idx, d_idx, rows0, rows1, acc, sem0, sem1) = refs
    c = lax.axis_index("c")
    s = lax.axis_index("s")
    tid = c * NS + s
    row0 = s * RPT

    pltpu.sync_copy(src.at[tid], s_idx)
    pltpu.sync_copy(dst.at[tid], d_idx)
    pltpu.sync_copy(zrows, acc.at[pl.ds(row0, RPT)])
    if with_deg:
      pltpu.sync_copy(ones, ones_v)
      pltpu.sync_copy(dzer, dacc.at[pl.ds(row0, RPT)])
    plsc.subcore_barrier()

    def scat(j, r):
      pltpu.sync_copy(r, acc.at[d_idx.at[j]], add=True)
      if with_deg:
        pltpu.sync_copy(ones_v, dacc.at[d_idx.at[j]], add=True)

    def chunkpair(i, carry):
      j0 = 2 * i
      g0 = pltpu.async_copy(p.at[s_idx.at[j0]], rows0, sem0)
      g1 = pltpu.async_copy(p.at[s_idx.at[j0 + 1]], rows1, sem1)
      g0.wait()
      scat(j0, rows0)
      g1.wait()
      scat(j0 + 1, rows1)
      return carry

    lax.fori_loop(0, NCHUNK // 2, chunkpair, 0)
    if NCHUNK % 2:
      pltpu.async_copy(p.at[s_idx.at[NCHUNK - 1]], rows0, sem0).wait()
      scat(NCHUNK - 1, rows0)
    plsc.subcore_barrier()

    pltpu.sync_copy(acc.at[pl.ds(row0, RPT)], agg_out.at[c, pl.ds(row0, RPT)])
    if with_deg:
      pltpu.sync_copy(dacc.at[pl.ds(row0, RPT)], deg_out.at[c, pl.ds(row0, RPT)])

  return pl.kernel(body, out_type=tuple(out_type), mesh=mesh,
                   scratch_types=scratch,
                   compiler_params=pltpu.CompilerParams(
                       use_tc_tiling_on_sc=tc_tiling))


def _entry_body(x, wn, ws, b, p, s):
  h = x[...]
  p[...] = jnp.dot(h, wn[...], preferred_element_type=jnp.float32)
  s[...] = jnp.dot(h, ws[...], preferred_element_type=jnp.float32) + b[...]


def _mid_body(sp, agg, deg, wn, ws, b, p, s):
  d = jnp.maximum(deg[:, 0:1] + deg[:, 1:2], 1.0)
  h = jnp.maximum(sp[...] + (agg[0] + agg[1]) / d, 0.0)
  p[...] = jnp.dot(h, wn[...], preferred_element_type=jnp.float32)
  s[...] = jnp.dot(h, ws[...], preferred_element_type=jnp.float32) + b[...]


def _fin_body(sp, agg, deg, out):
  d = jnp.maximum(deg[:, 0:1] + deg[:, 1:2], 1.0)
  out[...] = sp[...] + (agg[0] + agg[1]) / d


def _rows(w):
  return pl.BlockSpec((B_R, w), lambda i: (i, 0))


def _full(shape):
  return pl.BlockSpec(shape, lambda i: tuple(0 for _ in shape))


_GRID = N_PAD // B_R


def _entry_call(x, wn, ws, b, w_out):
  return pl.pallas_call(
      _entry_body,
      grid=(_GRID,),
      in_specs=[_rows(128), _full((128, w_out)), _full((128, w_out)),
                _full((1, w_out))],
      out_specs=[_rows(w_out), _rows(w_out)],
      out_shape=[jax.ShapeDtypeStruct((N_PAD, w_out), jnp.float32)] * 2,
  )(x, wn, ws, b)


def _mid_call(sp, agg, deg, wn, ws, b, w_in, w_out):
  return pl.pallas_call(
      _mid_body,
      grid=(_GRID,),
      in_specs=[_rows(w_in),
                pl.BlockSpec((NC, B_R, w_in), lambda i: (0, i, 0)),
                pl.BlockSpec((B_R, NC), lambda i: (i, 0)),
                _full((w_in, w_out)), _full((w_in, w_out)), _full((1, w_out))],
      out_specs=[_rows(w_out), _rows(w_out)],
      out_shape=[jax.ShapeDtypeStruct((N_PAD, w_out), jnp.float32)] * 2,
  )(sp, agg, deg, wn, ws, b)


def _fin_call(sp, agg, deg, w):
  return pl.pallas_call(
      _fin_body,
      grid=(_GRID,),
      in_specs=[_rows(w),
                pl.BlockSpec((NC, B_R, w), lambda i: (0, i, 0)),
                pl.BlockSpec((B_R, NC), lambda i: (i, 0))],
      out_specs=_rows(w),
      out_shape=jax.ShapeDtypeStruct((N_PAD, w), jnp.float32),
  )(sp, agg, deg)


def kernel(x, edge_index, Ws0, Wn0, b0, Ws1, Wn1, b1, Ws2, Wn2, b2):
  src = edge_index[0].reshape(NT, NCHUNK, K)
  dst = edge_index[1].reshape(NT, NCHUNK, K)
  xp = jnp.concatenate([x, jnp.zeros((N_PAD - N, D), jnp.float32)], axis=0)

  z128 = jnp.zeros((RPT, 128), jnp.float32)
  z64 = jnp.zeros((RPT, 64), jnp.float32)
  dzer = jnp.zeros((RPT,), jnp.float32)
  ones_k = jnp.ones((K,), jnp.float32)

  agg128_deg = _make_agg(128, True, tc_tiling=False)
  agg128 = _make_agg(128, False, tc_tiling=False)
  agg64 = _make_agg(64, False, tc_tiling=False)

  p0, s0 = _entry_call(xp, Wn0, Ws0, b0.reshape(1, -1), 128)
  agg0, degp = agg128_deg(z128, src, dst, p0, ones_k, dzer)
  degT = degp.T

  p1, s1 = _mid_call(s0, agg0, degT, Wn1, Ws1, b1.reshape(1, -1), 128, 128)
  (agg1,) = agg128(z128, src, dst, p1)

  p2, s2 = _mid_call(s1, agg1, degT, Wn2, Ws2, b2.reshape(1, -1), 128, 64)
  (agg2,) = agg64(z64, src, dst, p2)

  out = _fin_call(s2, agg2, degT, 64)
  return out[:N]


# speedup vs baseline: 2.6817x; 1.0866x over previous
import functools

import jax
import jax.numpy as jnp
from jax import lax
from jax.experimental import pallas as pl
from jax.experimental.pallas import tpu as pltpu
from jax.experimental.pallas import tpu_sc as plsc

N = 10000
E = 320000
D = 128
H = 128
C = 64

NC = 2
NS = 16
NT = NC * NS
EPT = E // NT
K = 80
NCHUNK = EPT // K

N_PAD = 10240
RPT = N_PAD // NS
B_R = 1024


def _make_agg(w, with_deg, tc_tiling=True):
  mesh = plsc.VectorSubcoreMesh(core_axis_name="c", subcore_axis_name="s")
  out_type = [jax.ShapeDtypeStruct((NC, N_PAD, w), jnp.float32)]
  scratch = [
      pltpu.VMEM((NCHUNK, K), jnp.int32),
      pltpu.VMEM((NCHUNK, K), jnp.int32),
      pltpu.VMEM((K, w), jnp.float32),
      pltpu.VMEM((K, w), jnp.float32),
      pltpu.VMEM_SHARED((N_PAD, w), jnp.float32),
      pltpu.SemaphoreType.DMA,
      pltpu.SemaphoreType.DMA,
  ]
  if with_deg:
    out_type.append(jax.ShapeDtypeStruct((NC, N_PAD), jnp.float32))
    scratch += [
        pltpu.VMEM((K,), jnp.float32),
        pltpu.VMEM_SHARED((N_PAD,), jnp.float32),
    ]

  def body(*refs):
    if with_deg:
      (zrows, src, dst, p, ones, dzer, agg_out, deg_out,
       s_idx, d_idx, rows0, rows1, acc, sem0, sem1, ones_v, dacc) = refs
    else:
      (zrows, src, dst, p, agg_out,
       s_idx, d_idx, rows0, rows1, acc, sem0, sem1) = refs
    c = lax.axis_index("c")
    s = lax.axis_index("s")
    tid = c * NS + s
    row0 = s * RPT

    pltpu.sync_copy(src.at[tid], s_idx)
    pltpu.sync_copy(dst.at[tid], d_idx)
    pltpu.sync_copy(zrows, acc.at[pl.ds(row0, RPT)])
    if with_deg:
      pltpu.sync_copy(ones, ones_v)
      pltpu.sync_copy(dzer, dacc.at[pl.ds(row0, RPT)])
    plsc.subcore_barrier()

    def scat(j, r):
      pltpu.sync_copy(r, acc.at[d_idx.at[j]], add=True)
      if with_deg:
        pltpu.sync_copy(ones_v, dacc.at[d_idx.at[j]], add=True)

    U = 5
    buf = ((rows0, sem0), (rows1, sem1))

    def gath(j, b):
      r, sm = buf[b % 2]
      return pltpu.async_copy(p.at[s_idx.at[j]], r, sm)

    def chunks(i, carry):
      j0 = U * i
      g = [gath(j0, 0), gath(j0 + 1, 1)]
      for k in range(U):
        g[k].wait()
        scat(j0 + k, buf[k % 2][0])
        if k + 2 < U:
          g.append(gath(j0 + k + 2, k))
      return carry

    lax.fori_loop(0, NCHUNK // U, chunks, 0)
    plsc.subcore_barrier()

    pltpu.sync_copy(acc.at[pl.ds(row0, RPT)], agg_out.at[c, pl.ds(row0, RPT)])
    if with_deg:
      pltpu.sync_copy(dacc.at[pl.ds(row0, RPT)], deg_out.at[c, pl.ds(row0, RPT)])

  return pl.kernel(body, out_type=tuple(out_type), mesh=mesh,
                   scratch_types=scratch,
                   compiler_params=pltpu.CompilerParams(
                       use_tc_tiling_on_sc=tc_tiling))


def _entry_body(x, wn, ws, b, p, s):
  h = x[...]
  p[...] = jnp.dot(h, wn[...], preferred_element_type=jnp.float32)
  s[...] = jnp.dot(h, ws[...], preferred_element_type=jnp.float32) + b[...]


def _mid_body(sp, agg, deg, wn, ws, b, p, s):
  d = jnp.maximum(deg[:, 0:1] + deg[:, 1:2], 1.0)
  h = jnp.maximum(sp[...] + (agg[0] + agg[1]) / d, 0.0)
  p[...] = jnp.dot(h, wn[...], preferred_element_type=jnp.float32)
  s[...] = jnp.dot(h, ws[...], preferred_element_type=jnp.float32) + b[...]


def _fin_body(sp, agg, deg, out):
  d = jnp.maximum(deg[:, 0:1] + deg[:, 1:2], 1.0)
  out[...] = sp[...] + (agg[0] + agg[1]) / d


def _rows(w):
  return pl.BlockSpec((B_R, w), lambda i: (i, 0))


def _full(shape):
  return pl.BlockSpec(shape, lambda i: tuple(0 for _ in shape))


_GRID = N_PAD // B_R


def _entry_call(x, wn, ws, b, w_out):
  return pl.pallas_call(
      _entry_body,
      grid=(_GRID,),
      in_specs=[_rows(128), _full((128, w_out)), _full((128, w_out)),
                _full((1, w_out))],
      out_specs=[_rows(w_out), _rows(w_out)],
      out_shape=[jax.ShapeDtypeStruct((N_PAD, w_out), jnp.float32)] * 2,
  )(x, wn, ws, b)


def _mid_call(sp, agg, deg, wn, ws, b, w_in, w_out):
  return pl.pallas_call(
      _mid_body,
      grid=(_GRID,),
      in_specs=[_rows(w_in),
                pl.BlockSpec((NC, B_R, w_in), lambda i: (0, i, 0)),
                pl.BlockSpec((B_R, NC), lambda i: (i, 0)),
                _full((w_in, w_out)), _full((w_in, w_out)), _full((1, w_out))],
      out_specs=[_rows(w_out), _rows(w_out)],
      out_shape=[jax.ShapeDtypeStruct((N_PAD, w_out), jnp.float32)] * 2,
  )(sp, agg, deg, wn, ws, b)


def _fin_call(sp, agg, deg, w):
  return pl.pallas_call(
      _fin_body,
      grid=(_GRID,),
      in_specs=[_rows(w),
                pl.BlockSpec((NC, B_R, w), lambda i: (0, i, 0)),
                pl.BlockSpec((B_R, NC), lambda i: (i, 0))],
      out_specs=_rows(w),
      out_shape=jax.ShapeDtypeStruct((N_PAD, w), jnp.float32),
  )(sp, agg, deg)


def kernel(x, edge_index, Ws0, Wn0, b0, Ws1, Wn1, b1, Ws2, Wn2, b2):
  src = edge_index[0].reshape(NT, NCHUNK, K)
  dst = edge_index[1].reshape(NT, NCHUNK, K)
  xp = jnp.concatenate([x, jnp.zeros((N_PAD - N, D), jnp.float32)], axis=0)

  z128 = jnp.zeros((RPT, 128), jnp.float32)
  z64 = jnp.zeros((RPT, 64), jnp.float32)
  dzer = jnp.zeros((RPT,), jnp.float32)
  ones_k = jnp.ones((K,), jnp.float32)

  agg128_deg = _make_agg(128, True, tc_tiling=False)
  agg128 = _make_agg(128, False, tc_tiling=False)
  agg64 = _make_agg(64, False, tc_tiling=False)

  p0, s0 = _entry_call(xp, Wn0, Ws0, b0.reshape(1, -1), 128)
  agg0, degp = agg128_deg(z128, src, dst, p0, ones_k, dzer)
  degT = degp.T

  p1, s1 = _mid_call(s0, agg0, degT, Wn1, Ws1, b1.reshape(1, -1), 128, 128)
  (agg1,) = agg128(z128, src, dst, p1)

  p2, s2 = _mid_call(s1, agg1, degT, Wn2, Ws2, b2.reshape(1, -1), 128, 64)
  (agg2,) = agg64(z64, src, dst, p2)

  out = _fin_call(s2, agg2, degT, 64)
  return out[:N]
